# baseline (speedup 1.0000x reference)
import functools

import jax
import jax.numpy as jnp
from jax import lax
from jax.experimental import pallas as pl
from jax.experimental.pallas import tpu as pltpu
from jax.experimental.pallas import tpu_sc as plsc

_VOCAB = 100000
_D = 64
_S = 200
_B = 4096

_NC = 2
_NS = 16
_L = 16
_NW = _NC * _NS
_BPW = _B // _NW
_G = _BPW // _L
_CH = 40

_ROW_BLK = 1024


def _tw_body(tbl_ref, w_ref, b_ref, t_ref):
    t_ref[...] = jnp.sum(tbl_ref[...] * w_ref[...], axis=1,
                         keepdims=True) + b_ref[0]


def _project_table(table, W, b):
    grid = (_VOCAB + _ROW_BLK - 1) // _ROW_BLK
    t2d = pl.pallas_call(
        _tw_body,
        grid=(grid,),
        in_specs=[
            pl.BlockSpec((_ROW_BLK, _D), lambda i: (i, 0)),
            pl.BlockSpec((1, _D), lambda i: (0, 0)),
            pl.BlockSpec(memory_space=pltpu.SMEM),
        ],
        out_specs=pl.BlockSpec((_ROW_BLK, 1), lambda i: (i, 0)),
        out_shape=jax.ShapeDtypeStruct((_VOCAB, 1), jnp.float32),
    )(table, W.reshape(1, _D), b)
    return t2d.reshape(_VOCAB)


def _rsqrt(z):
    i = lax.bitcast_convert_type(z, jnp.int32)
    y = lax.bitcast_convert_type(
        jnp.int32(0x5F3759DF) - lax.shift_right_arithmetic(i, 1), jnp.float32)
    for _ in range(3):
        y = y * (1.5 - 0.5 * z * y * y)
    return y


def _sc_body(q1_hbm, q2_hbm, t_hbm, out_hbm, t_v, q1_v, q2_v, st_v):
    wid = lax.axis_index("s") * _NC + lax.axis_index("c")
    base = wid * _BPW

    pltpu.sync_copy(t_hbm, t_v)

    zero = jnp.zeros((_L,), jnp.float32)
    accs = (zero,) * (3 * _G)

    for c in range(_S // _CH):
        pltpu.sync_copy(q1_hbm.at[pl.ds(c * _CH, _CH), pl.ds(base, _BPW)],
                        q1_v)
        pltpu.sync_copy(q2_hbm.at[pl.ds(c * _CH, _CH), pl.ds(base, _BPW)],
                        q2_v)

        def body(s, carry, q1_v=q1_v, q2_v=q2_v, t_v=t_v):
            new = list(carry)
            for g in range(_G):
                i1 = q1_v[s, pl.ds(g * _L, _L)]
                i2 = q2_v[s, pl.ds(g * _L, _L)]
                v1 = plsc.load_gather(t_v, [i1])
                v2 = plsc.load_gather(t_v, [i2])
                new[3 * g] = new[3 * g] + v1 * v2
                new[3 * g + 1] = new[3 * g + 1] + v1 * v1
                new[3 * g + 2] = new[3 * g + 2] + v2 * v2
            return tuple(new)

        accs = lax.fori_loop(0, _CH, body, accs)

    for g in range(_G):
        num = accs[3 * g]
        z = jnp.maximum(accs[3 * g + 1] * accs[3 * g + 2], 1e-28)
        denom = jnp.maximum(z * _rsqrt(z), 1e-8)
        cos = num / denom
        st_v[pl.ds(g * _L, _L)] = 1.0 / (1.0 + jnp.exp(-cos))

    pltpu.sync_copy(st_v, out_hbm.at[pl.ds(base, _BPW)])


def _sc_reduce(q1, q2, t):
    mesh = plsc.VectorSubcoreMesh(core_axis_name="c", subcore_axis_name="s")
    f = functools.partial(
        pl.kernel,
        out_type=jax.ShapeDtypeStruct((_B,), jnp.float32),
        mesh=mesh,
        scratch_types=[
            pltpu.VMEM((_VOCAB,), jnp.float32),
            pltpu.VMEM((_CH, _BPW), jnp.int32),
            pltpu.VMEM((_CH, _BPW), jnp.int32),
            pltpu.VMEM((_BPW,), jnp.float32),
        ],
        compiler_params=pltpu.CompilerParams(needs_layout_passes=False),
    )(_sc_body)
    return f(q1, q2, t)


def kernel(question1, question2, table, W, b):
    t = _project_table(table, W, b)
    return _sc_reduce(question1, question2, t)


# speedup vs baseline: 1.3393x; 1.3393x over previous
import functools

import jax
import jax.numpy as jnp
from jax import lax
from jax.experimental import pallas as pl
from jax.experimental.pallas import tpu as pltpu
from jax.experimental.pallas import tpu_sc as plsc

_VOCAB = 100000
_D = 64
_S = 200
_B = 4096

_NC = 2
_NS = 16
_L = 16
_NW = _NC * _NS
_BPW = _B // _NW
_G = _BPW // _L
_CH = 40

_ROW_BLK = 16384


def _tw_body(tbl_ref, w_ref, b_ref, t_ref):
    t_ref[...] = jnp.sum(tbl_ref[...] * w_ref[...], axis=1,
                         keepdims=True) + b_ref[0]


def _project_table(table, W, b):
    grid = (_VOCAB + _ROW_BLK - 1) // _ROW_BLK
    t2d = pl.pallas_call(
        _tw_body,
        grid=(grid,),
        in_specs=[
            pl.BlockSpec((_ROW_BLK, _D), lambda i: (i, 0)),
            pl.BlockSpec((1, _D), lambda i: (0, 0)),
            pl.BlockSpec(memory_space=pltpu.SMEM),
        ],
        out_specs=pl.BlockSpec((_ROW_BLK, 1), lambda i: (i, 0)),
        out_shape=jax.ShapeDtypeStruct((_VOCAB, 1), jnp.float32),
    )(table, W.reshape(1, _D), b)
    return t2d.reshape(_VOCAB)


def _rsqrt(z):
    i = lax.bitcast_convert_type(z, jnp.int32)
    y = lax.bitcast_convert_type(
        jnp.int32(0x5F3759DF) - lax.shift_right_arithmetic(i, 1), jnp.float32)
    for _ in range(3):
        y = y * (1.5 - 0.5 * z * y * y)
    return y


def _sc_body(q1_hbm, q2_hbm, t_hbm, out_hbm, t_v, q1_v, q2_v, st_v):
    wid = lax.axis_index("s") * _NC + lax.axis_index("c")
    base = wid * _BPW

    pltpu.sync_copy(t_hbm, t_v)

    zero = jnp.zeros((_L,), jnp.float32)
    accs = (zero,) * (3 * _G)

    for c in range(_S // _CH):
        pltpu.sync_copy(q1_hbm.at[pl.ds(c * _CH, _CH), pl.ds(base, _BPW)],
                        q1_v)
        pltpu.sync_copy(q2_hbm.at[pl.ds(c * _CH, _CH), pl.ds(base, _BPW)],
                        q2_v)

        def body(s, carry, q1_v=q1_v, q2_v=q2_v, t_v=t_v):
            new = list(carry)
            for g in range(_G):
                i1 = q1_v[s, pl.ds(g * _L, _L)]
                i2 = q2_v[s, pl.ds(g * _L, _L)]
                v1 = plsc.load_gather(t_v, [i1])
                v2 = plsc.load_gather(t_v, [i2])
                new[3 * g] = new[3 * g] + v1 * v2
                new[3 * g + 1] = new[3 * g + 1] + v1 * v1
                new[3 * g + 2] = new[3 * g + 2] + v2 * v2
            return tuple(new)

        accs = lax.fori_loop(0, _CH, body, accs)

    for g in range(_G):
        num = accs[3 * g]
        z = jnp.maximum(accs[3 * g + 1] * accs[3 * g + 2], 1e-28)
        denom = jnp.maximum(z * _rsqrt(z), 1e-8)
        cos = num / denom
        st_v[pl.ds(g * _L, _L)] = 1.0 / (1.0 + jnp.exp(-cos))

    pltpu.sync_copy(st_v, out_hbm.at[pl.ds(base, _BPW)])


def _sc_reduce(q1, q2, t):
    mesh = plsc.VectorSubcoreMesh(core_axis_name="c", subcore_axis_name="s")
    f = functools.partial(
        pl.kernel,
        out_type=jax.ShapeDtypeStruct((_B,), jnp.float32),
        mesh=mesh,
        scratch_types=[
            pltpu.VMEM((_VOCAB,), jnp.float32),
            pltpu.VMEM((_CH, _BPW), jnp.int32),
            pltpu.VMEM((_CH, _BPW), jnp.int32),
            pltpu.VMEM((_BPW,), jnp.float32),
        ],
        compiler_params=pltpu.CompilerParams(needs_layout_passes=False),
    )(_sc_body)
    return f(q1, q2, t)


def kernel(question1, question2, table, W, b):
    t = _project_table(table, W, b)
    return _sc_reduce(question1, question2, t)
